# baseline (speedup 1.0000x reference)
import dataclasses
import functools

import jax
import jax.numpy as jnp
from jax import lax
from jax.experimental import pallas as pl
from jax.experimental.pallas import tpu as pltpu
from jax.experimental.pallas import tpu_sc as plsc

N = 10000
NP = 10240
D = 128
E = 320000
NC, NS, L = 2, 16, 16
NW = NC * NS
EPT = 10240
EP = NW * EPT
RPT = NP // NS
DPT = NP // NW

C = 128
G = EPT // C
G2 = 2 * G
BLK = 16
BLK1 = 8
BLK2 = 8

_mesh = plsc.VectorSubcoreMesh(core_axis_name="c", subcore_axis_name="s")

_sc_params = pltpu.CompilerParams()
if "needs_layout_passes" in pltpu.CompilerParams.__dataclass_fields__:
    _sc_params = dataclasses.replace(_sc_params, needs_layout_passes=False)

_f32 = jnp.float32
_i32 = jnp.int32
_zero16 = functools.partial(jnp.zeros, (L,), _f32)


def _degdinv_body(dst_hbm, ew_hbm, out_hbm, idxb, valb, zb, shared, ssem):
    s = lax.axis_index("s")
    w = lax.axis_index("c") * NS + s

    @pl.loop(0, RPT, step=L)
    def _(i):
        zb[pl.ds(i, L)] = _zero16()

    pltpu.sync_copy(zb, shared.at[pl.ds(s * RPT, RPT)])
    plsc.subcore_barrier()

    @pl.loop(0, G2 // BLK)
    def _(b):
        pltpu.sync_copy(dst_hbm.at[s, pl.ds(b * BLK, BLK)], idxb)
        pltpu.sync_copy(ew_hbm.at[s, pl.ds(b * BLK, BLK)], valb)

        @pl.loop(0, BLK)
        def _(g):
            pltpu.async_copy(valb.at[g], shared.at[idxb.at[g]], ssem,
                             add=True)

        @pl.loop(0, BLK)
        def _(g):
            pltpu.make_async_copy(valb.at[g], shared.at[idxb.at[g]],
                                  ssem).wait()

    plsc.subcore_barrier()
    pltpu.sync_copy(shared.at[pl.ds(w * DPT, DPT)], zb.at[pl.ds(0, DPT)])

    @pl.loop(0, DPT, step=L)
    def _(i):
        sl = pl.ds(i, L)
        x = zb[sl] + 1.0
        ib = plsc.bitcast(x, _i32)
        magic = jnp.full((L,), 0x5F3759DF, _i32)
        y = plsc.bitcast(magic - lax.shift_right_logical(ib, 1), _f32)
        half = x * (-0.5)
        for _ in range(3):
            y = y * (half * y * y + 1.5)
        zb[sl] = y

    pltpu.sync_copy(zb.at[pl.ds(0, DPT)], out_hbm.at[pl.ds(w * DPT, DPT)])


_degdinv = pl.kernel(_degdinv_body,
                     out_type=jax.ShapeDtypeStruct((NP,), _f32),
                     mesh=_mesh,
                     compiler_params=_sc_params,
                     scratch_types=[
                         pltpu.VMEM((BLK, C), _i32),
                         pltpu.VMEM((BLK, C), _f32),
                         pltpu.VMEM((RPT,), _f32),
                         pltpu.VMEM_SHARED((NP,), _f32),
                         pltpu.SemaphoreType.DMA,
                     ])


def _agg1_body(xw_hbm, dinv_hbm, src_hbm, dst_hbm, ew_hbm, out_hbm,
               sidxb, didxb, ewvb, dv0, dv1, rows0, rows1, shared,
               gsem0, gsem1, dsem0, dsem1, ssem0, ssem1):
    c = lax.axis_index("c")
    s = lax.axis_index("s")
    w = c * NS + s
    bufs = (rows0, rows1)
    dvs = (dv0, dv1)
    gsems = (gsem0, gsem1)
    dsems = (dsem0, dsem1)
    ssems = (ssem0, ssem1)

    @pl.loop(0, C)
    def _(r):
        for j in range(8):
            rows0[r, pl.ds(j * L, L)] = _zero16()

    base = s * RPT
    for k in range(RPT // C):
        pltpu.sync_copy(rows0, shared.at[pl.ds(base + k * C, C)])
    plsc.subcore_barrier()

    def gather(g, i):
        pltpu.async_copy(xw_hbm.at[sidxb.at[g]], bufs[i], gsems[i])
        pltpu.async_copy(dinv_hbm.at[sidxb.at[g]], dvs[i], dsems[i])

    def gather_wait(g, i):
        pltpu.make_async_copy(xw_hbm.at[sidxb.at[g]], bufs[i],
                              gsems[i]).wait()
        pltpu.make_async_copy(dinv_hbm.at[sidxb.at[g]], dvs[i],
                              dsems[i]).wait()

    def scatter(g, i):
        pltpu.async_copy(bufs[i], shared.at[didxb.at[g]], ssems[i], add=True)

    def scatter_wait(g, i):
        pltpu.make_async_copy(bufs[i], shared.at[didxb.at[g]],
                              ssems[i]).wait()

    def scale(g, i):
        buf = bufs[i]
        dv = dvs[i]

        @pl.loop(0, C // L)
        def _(q):
            qsl = pl.ds(q * L, L)
            ew16 = ewvb[g, qsl] * dv[qsl]
            for r16 in range(L):
                bv = lax.broadcast_in_dim(ew16[r16], (L,), ())
                r = q * L + r16
                for j in range(8):
                    sl = pl.ds(j * L, L)
                    buf[r, sl] = buf[r, sl] * bv

    @pl.loop(0, G // BLK1)
    def _(b):
        pltpu.sync_copy(src_hbm.at[w, pl.ds(b * BLK1, BLK1)], sidxb)
        pltpu.sync_copy(dst_hbm.at[w, pl.ds(b * BLK1, BLK1)], didxb)
        pltpu.sync_copy(ew_hbm.at[w, pl.ds(b * BLK1, BLK1)], ewvb)

        gather(0, 0)
        gather(1, 1)

        @pl.loop(0, BLK1, step=2)
        def _(g):
            gather_wait(g, 0)
            scale(g, 0)
            scatter(g, 0)

            gather_wait(g + 1, 1)
            scale(g + 1, 1)
            scatter(g + 1, 1)

            scatter_wait(g, 0)

            @pl.when(g + 2 < BLK1)
            def _():
                gather(g + 2, 0)

            scatter_wait(g + 1, 1)

            @pl.when(g + 3 < BLK1)
            def _():
                gather(g + 3, 1)

    plsc.subcore_barrier()
    for k in range(RPT // C):
        pltpu.sync_copy(shared.at[pl.ds(base + k * C, C)],
                        out_hbm.at[c, pl.ds(base + k * C, C)])


_agg1 = pl.kernel(_agg1_body,
                  out_type=jax.ShapeDtypeStruct((NC, NP, D), _f32),
                  mesh=_mesh,
                  scratch_types=(
                      [pltpu.VMEM((BLK1, C), _i32),
                       pltpu.VMEM((BLK1, C), _i32),
                       pltpu.VMEM((BLK1, C), _f32),
                       pltpu.VMEM((C,), _f32),
                       pltpu.VMEM((C,), _f32)]
                      + [pltpu.VMEM((C, D), _f32)] * 2
                      + [pltpu.VMEM_SHARED((NP, D), _f32)]
                      + [pltpu.SemaphoreType.DMA] * 6
                  ))


def _agg2_body(z_hbm, src_hbm, dst_hbm, ew_hbm, out_hbm,
               sidxb, didxb, ewvb, zg0, zg1, zb, shared,
               gsem0, gsem1, ssem0, ssem1):
    c = lax.axis_index("c")
    s = lax.axis_index("s")
    w = c * NS + s
    bufs = (zg0, zg1)
    gsems = (gsem0, gsem1)
    ssems = (ssem0, ssem1)

    @pl.loop(0, RPT, step=L)
    def _(i):
        zb[pl.ds(i, L)] = _zero16()

    pltpu.sync_copy(zb, shared.at[pl.ds(s * RPT, RPT)])
    plsc.subcore_barrier()

    def gather(g, i):
        pltpu.async_copy(z_hbm.at[sidxb.at[g]], bufs[i], gsems[i])

    def gather_wait(g, i):
        pltpu.make_async_copy(z_hbm.at[sidxb.at[g]], bufs[i],
                              gsems[i]).wait()

    def scatter(g, i):
        pltpu.async_copy(bufs[i], shared.at[didxb.at[g]], ssems[i], add=True)

    def scatter_wait(g, i):
        pltpu.make_async_copy(bufs[i], shared.at[didxb.at[g]],
                              ssems[i]).wait()

    def scale(g, i):
        buf = bufs[i]
        for j in range(C // L):
            sl = pl.ds(j * L, L)
            buf[sl] = buf[sl] * ewvb[g, sl]

    @pl.loop(0, G // BLK2)
    def _(b):
        pltpu.sync_copy(src_hbm.at[w, pl.ds(b * BLK2, BLK2)], sidxb)
        pltpu.sync_copy(dst_hbm.at[w, pl.ds(b * BLK2, BLK2)], didxb)
        pltpu.sync_copy(ew_hbm.at[w, pl.ds(b * BLK2, BLK2)], ewvb)

        gather(0, 0)
        gather(1, 1)

        @pl.loop(0, BLK2, step=2)
        def _(g):
            gather_wait(g, 0)
            scale(g, 0)
            scatter(g, 0)

            gather_wait(g + 1, 1)
            scale(g + 1, 1)
            scatter(g + 1, 1)

            scatter_wait(g, 0)

            @pl.when(g + 2 < BLK2)
            def _():
                gather(g + 2, 0)

            scatter_wait(g + 1, 1)

            @pl.when(g + 3 < BLK2)
            def _():
                gather(g + 3, 1)

    plsc.subcore_barrier()
    pltpu.sync_copy(shared.at[pl.ds(s * RPT, RPT)],
                    out_hbm.at[c, pl.ds(s * RPT, RPT)])


_agg2 = pl.kernel(_agg2_body,
                  out_type=jax.ShapeDtypeStruct((NC, NP), _f32),
                  mesh=_mesh,
                  compiler_params=_sc_params,
                  scratch_types=[
                      pltpu.VMEM((BLK2, C), _i32),
                      pltpu.VMEM((BLK2, C), _i32),
                      pltpu.VMEM((BLK2, C), _f32),
                      pltpu.VMEM((C,), _f32),
                      pltpu.VMEM((C,), _f32),
                      pltpu.VMEM((RPT,), _f32),
                      pltpu.VMEM_SHARED((NP,), _f32),
                      pltpu.SemaphoreType.DMA,
                      pltpu.SemaphoreType.DMA,
                      pltpu.SemaphoreType.DMA,
                      pltpu.SemaphoreType.DMA,
                  ])


def _mm1_b(x_ref, w_ref, o_ref):
    o_ref[...] = jnp.dot(x_ref[...], w_ref[...],
                         preferred_element_type=_f32,
                         precision=lax.Precision.HIGHEST)


def _mid_b(aggp_ref, xw_ref, dinv_ref, b1_ref, w2_ref, z_ref):
    dinv = dinv_ref[...]
    h = (aggp_ref[0] + aggp_ref[1] + xw_ref[...] * dinv) * dinv + b1_ref[...]
    h = jnp.maximum(h, 0.0)
    z_ref[...] = jnp.dot(h, w2_ref[...],
                         preferred_element_type=_f32,
                         precision=lax.Precision.HIGHEST) * dinv


def _fin_b(a2_ref, z_ref, dinv_ref, b2_ref, o_ref):
    t = (a2_ref[0] + a2_ref[1] + z_ref[...]) * dinv_ref[...] + b2_ref[0, 0]
    o_ref[...] = jax.nn.sigmoid(t)


def _tc(body, out_shape):
    return pl.pallas_call(body, out_shape=out_shape)


def kernel(x, edge_index, edge_weight, W1, b1, W2, b2):
    src = edge_index[0]
    dst = edge_index[1]
    pad = EP - E
    pad_idx = (jnp.arange(pad, dtype=_i32) * 97) % N
    src_p = jnp.concatenate([src, pad_idx])
    dst_p = jnp.concatenate([dst, pad_idx])
    ew_p = jnp.concatenate([edge_weight, jnp.zeros((pad,), _f32)])
    x_pad = jnp.pad(x, ((0, NP - N), (0, 0)))

    xw = _tc(_mm1_b, jax.ShapeDtypeStruct((NP, D), _f32))(x_pad, W1)
    dinv = _degdinv(dst_p.reshape(NS, G2, C), ew_p.reshape(NS, G2, C))

    aggp = _agg1(xw, dinv, src_p.reshape(NW, G, C), dst_p.reshape(NW, G, C),
                 ew_p.reshape(NW, G, C))

    dinv2 = dinv.reshape(NP, 1)
    z = _tc(_mid_b, jax.ShapeDtypeStruct((NP, 1), _f32))(
        aggp, xw, dinv2, b1.reshape(1, D), W2)

    agg2p = _agg2(z.reshape(NP), src_p.reshape(NW, G, C),
                  dst_p.reshape(NW, G, C), ew_p.reshape(NW, G, C))

    out = _tc(_fin_b, jax.ShapeDtypeStruct((NP, 1), _f32))(
        agg2p.reshape(NC, NP, 1), z, dinv2, b2.reshape(1, 1))
    return out[:N]


# speedup vs baseline: 1.0411x; 1.0411x over previous
import dataclasses
import functools

import jax
import jax.numpy as jnp
from jax import lax
from jax.experimental import pallas as pl
from jax.experimental.pallas import tpu as pltpu
from jax.experimental.pallas import tpu_sc as plsc

N = 10000
NP = 10240
D = 128
E = 320000
NC, NS, L = 2, 16, 16
NW = NC * NS
EPT = 10240
EP = NW * EPT
RPT = NP // NS

C = 128
G = EPT // C
C1 = 64
G1 = EPT // C1
BLK = 16
BLK2 = 8

_mesh = plsc.VectorSubcoreMesh(core_axis_name="c", subcore_axis_name="s")

_sc_params = pltpu.CompilerParams()
if "needs_layout_passes" in pltpu.CompilerParams.__dataclass_fields__:
    _sc_params = dataclasses.replace(_sc_params, needs_layout_passes=False)

_f32 = jnp.float32
_i32 = jnp.int32
_zero16 = functools.partial(jnp.zeros, (L,), _f32)


def _deg_body(dst_hbm, ew_hbm, out_hbm, idxb, valb, zb, shared, ssem):
    c = lax.axis_index("c")
    s = lax.axis_index("s")
    w = c * NS + s

    @pl.loop(0, RPT, step=L)
    def _(i):
        zb[pl.ds(i, L)] = _zero16()

    pltpu.sync_copy(zb, shared.at[pl.ds(s * RPT, RPT)])
    plsc.subcore_barrier()

    @pl.loop(0, G // BLK)
    def _(b):
        pltpu.sync_copy(dst_hbm.at[w, pl.ds(b * BLK, BLK)], idxb)
        pltpu.sync_copy(ew_hbm.at[w, pl.ds(b * BLK, BLK)], valb)

        @pl.loop(0, BLK)
        def _(g):
            pltpu.async_copy(valb.at[g], shared.at[idxb.at[g]], ssem,
                             add=True)

        @pl.loop(0, BLK)
        def _(g):
            pltpu.make_async_copy(valb.at[g], shared.at[idxb.at[g]],
                                  ssem).wait()

    plsc.subcore_barrier()
    pltpu.sync_copy(shared.at[pl.ds(s * RPT, RPT)],
                    out_hbm.at[c, pl.ds(s * RPT, RPT)])


_deg = pl.kernel(_deg_body,
                 out_type=jax.ShapeDtypeStruct((NC, NP), _f32),
                 mesh=_mesh,
                 scratch_types=[
                     pltpu.VMEM((BLK, C), _i32),
                     pltpu.VMEM((BLK, C), _f32),
                     pltpu.VMEM((RPT,), _f32),
                     pltpu.VMEM_SHARED((NP,), _f32),
                     pltpu.SemaphoreType.DMA,
                 ])


def _agg1_body(y_hbm, src_hbm, dst_hbm, ew_hbm, out_hbm,
               sidxb, didxb, ewvb, rows0, rows1, rows2, rows3, shared,
               gsem0, gsem1, gsem2, gsem3, ssem0, ssem1, ssem2, ssem3):
    c = lax.axis_index("c")
    s = lax.axis_index("s")
    w = c * NS + s
    bufs = (rows0, rows1, rows2, rows3)
    gsems = (gsem0, gsem1, gsem2, gsem3)
    ssems = (ssem0, ssem1, ssem2, ssem3)

    @pl.loop(0, C1)
    def _(r):
        for j in range(8):
            rows0[r, pl.ds(j * L, L)] = _zero16()

    base = s * RPT
    for k in range(RPT // C1):
        pltpu.sync_copy(rows0, shared.at[pl.ds(base + k * C1, C1)])
    plsc.subcore_barrier()

    def gather(g, i):
        pltpu.async_copy(y_hbm.at[sidxb.at[g]], bufs[i], gsems[i])

    def gather_wait(g, i):
        pltpu.make_async_copy(y_hbm.at[sidxb.at[g]], bufs[i],
                              gsems[i]).wait()

    def scatter(g, i):
        pltpu.async_copy(bufs[i], shared.at[didxb.at[g]], ssems[i], add=True)

    def scatter_wait(g, i):
        pltpu.make_async_copy(bufs[i], shared.at[didxb.at[g]],
                              ssems[i]).wait()

    def scale(g, i):
        buf = bufs[i]

        @pl.loop(0, C1 // L)
        def _(q):
            ew16 = ewvb[g, pl.ds(q * L, L)]
            for r16 in range(L):
                bv = lax.broadcast_in_dim(ew16[r16], (L,), ())
                r = q * L + r16
                for j in range(8):
                    sl = pl.ds(j * L, L)
                    buf[r, sl] = buf[r, sl] * bv

    @pl.loop(0, G1 // BLK)
    def _(b):
        pltpu.sync_copy(src_hbm.at[w, pl.ds(b * BLK, BLK)], sidxb)
        pltpu.sync_copy(dst_hbm.at[w, pl.ds(b * BLK, BLK)], didxb)
        pltpu.sync_copy(ew_hbm.at[w, pl.ds(b * BLK, BLK)], ewvb)

        for i in range(4):
            gather(i, i)

        @pl.loop(0, BLK, step=4)
        def _(g):
            gather_wait(g, 0)
            scale(g, 0)
            scatter(g, 0)

            gather_wait(g + 1, 1)
            scale(g + 1, 1)
            scatter(g + 1, 1)

            gather_wait(g + 2, 2)
            scale(g + 2, 2)
            scatter(g + 2, 2)

            scatter_wait(g, 0)

            @pl.when(g + 4 < BLK)
            def _():
                gather(g + 4, 0)

            gather_wait(g + 3, 3)
            scale(g + 3, 3)
            scatter(g + 3, 3)

            scatter_wait(g + 1, 1)

            @pl.when(g + 5 < BLK)
            def _():
                gather(g + 5, 1)

            scatter_wait(g + 2, 2)

            @pl.when(g + 6 < BLK)
            def _():
                gather(g + 6, 2)

            scatter_wait(g + 3, 3)

            @pl.when(g + 7 < BLK)
            def _():
                gather(g + 7, 3)

    plsc.subcore_barrier()
    for k in range(RPT // C1):
        pltpu.sync_copy(shared.at[pl.ds(base + k * C1, C1)],
                        out_hbm.at[c, pl.ds(base + k * C1, C1)])


_agg1 = pl.kernel(_agg1_body,
                  out_type=jax.ShapeDtypeStruct((NC, NP, D), _f32),
                  mesh=_mesh,
                  scratch_types=(
                      [pltpu.VMEM((BLK, C1), _i32),
                       pltpu.VMEM((BLK, C1), _i32),
                       pltpu.VMEM((BLK, C1), _f32)]
                      + [pltpu.VMEM((C1, D), _f32)] * 4
                      + [pltpu.VMEM_SHARED((NP, D), _f32)]
                      + [pltpu.SemaphoreType.DMA] * 8
                  ))


def _agg2_body(z_hbm, src_hbm, dst_hbm, ew_hbm, out_hbm,
               sidxb, didxb, ewvb, zg0, zg1, zb, shared,
               gsem0, gsem1, ssem0, ssem1):
    c = lax.axis_index("c")
    s = lax.axis_index("s")
    w = c * NS + s
    bufs = (zg0, zg1)
    gsems = (gsem0, gsem1)
    ssems = (ssem0, ssem1)

    @pl.loop(0, RPT, step=L)
    def _(i):
        zb[pl.ds(i, L)] = _zero16()

    pltpu.sync_copy(zb, shared.at[pl.ds(s * RPT, RPT)])
    plsc.subcore_barrier()

    def gather(g, i):
        pltpu.async_copy(z_hbm.at[sidxb.at[g]], bufs[i], gsems[i])

    def gather_wait(g, i):
        pltpu.make_async_copy(z_hbm.at[sidxb.at[g]], bufs[i],
                              gsems[i]).wait()

    def scatter(g, i):
        pltpu.async_copy(bufs[i], shared.at[didxb.at[g]], ssems[i], add=True)

    def scatter_wait(g, i):
        pltpu.make_async_copy(bufs[i], shared.at[didxb.at[g]],
                              ssems[i]).wait()

    def scale(g, i):
        buf = bufs[i]
        for j in range(C // L):
            sl = pl.ds(j * L, L)
            buf[sl] = buf[sl] * ewvb[g, sl]

    @pl.loop(0, G // BLK2)
    def _(b):
        pltpu.sync_copy(src_hbm.at[w, pl.ds(b * BLK2, BLK2)], sidxb)
        pltpu.sync_copy(dst_hbm.at[w, pl.ds(b * BLK2, BLK2)], didxb)
        pltpu.sync_copy(ew_hbm.at[w, pl.ds(b * BLK2, BLK2)], ewvb)

        gather(0, 0)
        gather(1, 1)

        @pl.loop(0, BLK2, step=2)
        def _(g):
            gather_wait(g, 0)
            scale(g, 0)
            scatter(g, 0)

            gather_wait(g + 1, 1)
            scale(g + 1, 1)
            scatter(g + 1, 1)

            scatter_wait(g, 0)

            @pl.when(g + 2 < BLK2)
            def _():
                gather(g + 2, 0)

            scatter_wait(g + 1, 1)

            @pl.when(g + 3 < BLK2)
            def _():
                gather(g + 3, 1)

    plsc.subcore_barrier()
    pltpu.sync_copy(shared.at[pl.ds(s * RPT, RPT)],
                    out_hbm.at[c, pl.ds(s * RPT, RPT)])


_agg2 = pl.kernel(_agg2_body,
                  out_type=jax.ShapeDtypeStruct((NC, NP), _f32),
                  mesh=_mesh,
                  compiler_params=_sc_params,
                  scratch_types=[
                      pltpu.VMEM((BLK2, C), _i32),
                      pltpu.VMEM((BLK2, C), _i32),
                      pltpu.VMEM((BLK2, C), _f32),
                      pltpu.VMEM((C,), _f32),
                      pltpu.VMEM((C,), _f32),
                      pltpu.VMEM((RPT,), _f32),
                      pltpu.VMEM_SHARED((NP,), _f32),
                      pltpu.SemaphoreType.DMA,
                      pltpu.SemaphoreType.DMA,
                      pltpu.SemaphoreType.DMA,
                      pltpu.SemaphoreType.DMA,
                  ])


def _mm1_b(x_ref, w_ref, o_ref):
    o_ref[...] = jnp.dot(x_ref[...], w_ref[...],
                         preferred_element_type=_f32,
                         precision=lax.Precision.HIGHEST)


def _prep_b(degp_ref, xw_ref, dinv_ref, y_ref):
    deg = degp_ref[0] + degp_ref[1] + 1.0
    dinv = lax.rsqrt(deg)
    dinv_ref[...] = dinv
    y_ref[...] = xw_ref[...] * dinv


def _mid_b(aggp_ref, y_ref, dinv_ref, b1_ref, w2_ref, z_ref):
    h = (aggp_ref[0] + aggp_ref[1] + y_ref[...]) * dinv_ref[...] + b1_ref[...]
    h = jnp.maximum(h, 0.0)
    z_ref[...] = jnp.dot(h, w2_ref[...],
                         preferred_element_type=_f32,
                         precision=lax.Precision.HIGHEST) * dinv_ref[...]


def _fin_b(a2_ref, z_ref, dinv_ref, b2_ref, o_ref):
    t = (a2_ref[0] + a2_ref[1] + z_ref[...]) * dinv_ref[...] + b2_ref[0, 0]
    o_ref[...] = jax.nn.sigmoid(t)


def _tc(body, out_shape):
    return pl.pallas_call(body, out_shape=out_shape)


def kernel(x, edge_index, edge_weight, W1, b1, W2, b2):
    src = edge_index[0]
    dst = edge_index[1]
    pad = EP - E
    pad_idx = (jnp.arange(pad, dtype=_i32) * 97) % N
    src_p = jnp.concatenate([src, pad_idx])
    dst_p = jnp.concatenate([dst, pad_idx])
    ew_p = jnp.concatenate([edge_weight, jnp.zeros((pad,), _f32)])
    x_pad = jnp.pad(x, ((0, NP - N), (0, 0)))

    xw = _tc(_mm1_b, jax.ShapeDtypeStruct((NP, D), _f32))(x_pad, W1)
    degp = _deg(dst_p.reshape(NW, G, C), ew_p.reshape(NW, G, C))

    dinv, y = _tc(_prep_b, (jax.ShapeDtypeStruct((NP, 1), _f32),
                            jax.ShapeDtypeStruct((NP, D), _f32)))(
        degp.reshape(NC, NP, 1), xw)

    aggp = _agg1(y, src_p.reshape(NW, G1, C1), dst_p.reshape(NW, G1, C1),
                 ew_p.reshape(NW, G1, C1))

    z = _tc(_mid_b, jax.ShapeDtypeStruct((NP, 1), _f32))(
        aggp, y, dinv, b1.reshape(1, D), W2)

    agg2p = _agg2(z.reshape(NP), src_p.reshape(NW, G, C),
                  dst_p.reshape(NW, G, C), ew_p.reshape(NW, G, C))

    out = _tc(_fin_b, jax.ShapeDtypeStruct((NP, 1), _f32))(
        agg2p.reshape(NC, NP, 1), z, dinv, b2.reshape(1, 1))
    return out[:N]
